# Initial kernel scaffold; baseline (speedup 1.0000x reference)
import jax
import jax.numpy as jnp
from jax.experimental import pallas as pl


def kernel(x, Wg, bg, W1, b1, W2, b2):
    raise NotImplementedError("write your pallas kernel here")


# speedup vs baseline: 1.0701x; 1.0701x over previous
import functools

import jax
import jax.numpy as jnp
from jax import lax
from jax.experimental import pallas as pl
from jax.experimental.pallas import tpu as pltpu
from jax.experimental.pallas import tpu_sc as plsc

_D = 1024
_F = 4096
_E = 8
_K = 2
_T = 4096
_A = _T * _K
_B = 256
_NB = _A // _B + _E
_NP = _NB * _B
_FT = 1024
_NF = _F // _FT
_TB = 512
_NC = 2
_NS = 16
_NW = _NC * _NS
_L = 16


def _gate_body(x_ref, wg_ref, bg_ref, g_ref, s_ref):
    logits = jnp.dot(x_ref[...], wg_ref[...],
                     preferred_element_type=jnp.float32) + bg_ref[...]
    iota = lax.broadcasted_iota(jnp.int32, logits.shape, 1)
    m1 = jnp.max(logits, axis=1, keepdims=True)
    i1 = jnp.min(jnp.where(logits == m1, iota, _E), axis=1, keepdims=True)
    sel1 = iota == i1
    neg = jnp.float32(float("-inf"))
    l2 = jnp.where(sel1, neg, logits)
    m2 = jnp.max(l2, axis=1, keepdims=True)
    i2 = jnp.min(jnp.where(l2 == m2, iota, _E), axis=1, keepdims=True)
    sel2 = iota == i2
    e21 = jnp.exp(m2 - m1)
    w1 = 1.0 / (1.0 + e21)
    w2 = e21 / (1.0 + e21)
    g_ref[...] = jnp.where(sel1, w1, 0.0) + jnp.where(sel2, w2, 0.0)
    s_ref[...] = (sel1 | sel2).astype(jnp.int32)


def _gate(x_flat, Wg, bg2d):
    return pl.pallas_call(
        _gate_body,
        grid=(_T // _TB,),
        in_specs=[
            pl.BlockSpec((_TB, _D), lambda i: (i, 0)),
            pl.BlockSpec((_D, _E), lambda i: (0, 0)),
            pl.BlockSpec((1, _E), lambda i: (0, 0)),
        ],
        out_specs=[
            pl.BlockSpec((_TB, _E), lambda i: (i, 0)),
            pl.BlockSpec((_TB, _E), lambda i: (i, 0)),
        ],
        out_shape=[
            jax.ShapeDtypeStruct((_T, _E), jnp.float32),
            jax.ShapeDtypeStruct((_T, _E), jnp.int32),
        ],
    )(x_flat, Wg, bg2d)


def _ffn_body(be_ref, xs_ref, w1_ref, b1_ref, w2_ref, b2_ref, wc_ref,
              ys_ref, acc_ref):
    f = pl.program_id(1)

    @pl.when(f == 0)
    def _init():
        acc_ref[...] = jnp.zeros_like(acc_ref)

    h = jnp.maximum(
        jnp.dot(xs_ref[...], w1_ref[0],
                preferred_element_type=jnp.float32) + b1_ref[0], 0.0)
    acc_ref[...] += jnp.dot(h, w2_ref[0], preferred_element_type=jnp.float32)

    @pl.when(f == _NF - 1)
    def _fin():
        ys_ref[...] = (acc_ref[...] + b2_ref[0]) * wc_ref[...][:, 0:1]


def _ffn(block_expert, xs, W1, b1r, W2, b2r, w_mat):
    grid_spec = pltpu.PrefetchScalarGridSpec(
        num_scalar_prefetch=1,
        grid=(_NB, _NF),
        in_specs=[
            pl.BlockSpec((_B, _D), lambda b, f, be: (b, 0)),
            pl.BlockSpec((1, _D, _FT), lambda b, f, be: (be[b], 0, f)),
            pl.BlockSpec((1, 1, _FT), lambda b, f, be: (be[b], 0, f)),
            pl.BlockSpec((1, _FT, _D), lambda b, f, be: (be[b], f, 0)),
            pl.BlockSpec((1, 1, _D), lambda b, f, be: (be[b], 0, 0)),
            pl.BlockSpec((_B, 128), lambda b, f, be: (b, 0)),
        ],
        out_specs=pl.BlockSpec((_B, _D), lambda b, f, be: (b, 0)),
        scratch_shapes=[pltpu.VMEM((_B, _D), jnp.float32)],
    )
    return pl.pallas_call(
        _ffn_body,
        grid_spec=grid_spec,
        out_shape=jax.ShapeDtypeStruct((_NP, _D), jnp.float32),
    )(block_expert, xs, W1, b1r, W2, b2r, w_mat)


def _sc_gather(x_flat, src_tok):
    rpw = _NP // _NW
    cs = 64
    mesh = plsc.VectorSubcoreMesh(core_axis_name="c", subcore_axis_name="s")

    @functools.partial(
        pl.kernel, mesh=mesh,
        out_type=jax.ShapeDtypeStruct((_NP, _D), jnp.float32),
        scratch_types=[
            pltpu.VMEM((cs,), jnp.int32),
            pltpu.VMEM((cs, _D), jnp.float32),
            pltpu.SemaphoreType.DMA,
        ],
    )
    def k(x_hbm, idx_hbm, out_hbm, idx_v, rows_v, sem):
        wid = lax.axis_index("s") * _NC + lax.axis_index("c")
        base = wid * rpw

        def body(i, carry):
            start = base + i * cs
            pltpu.sync_copy(idx_hbm.at[pl.ds(start, cs)], idx_v)
            pltpu.async_copy(x_hbm.at[idx_v], rows_v, sem).wait()
            pltpu.sync_copy(rows_v, out_hbm.at[pl.ds(start, cs)])
            return carry

        lax.fori_loop(0, rpw // cs, body, 0)

    return k(x_flat, src_tok)


def _sc_combine(ys, p0, p1):
    tpw = _T // _NW
    cs = 32
    mesh = plsc.VectorSubcoreMesh(core_axis_name="c", subcore_axis_name="s")

    @functools.partial(
        pl.kernel, mesh=mesh,
        out_type=jax.ShapeDtypeStruct((_T, _D), jnp.float32),
        scratch_types=[
            pltpu.VMEM((cs,), jnp.int32),
            pltpu.VMEM((cs,), jnp.int32),
            pltpu.VMEM((cs, _D), jnp.float32),
            pltpu.VMEM((cs, _D), jnp.float32),
            pltpu.SemaphoreType.DMA,
        ],
    )
    def k(ys_hbm, p0_hbm, p1_hbm, out_hbm, i0, i1, r0, r1, sem):
        wid = lax.axis_index("s") * _NC + lax.axis_index("c")
        base = wid * tpw

        def body(c, carry):
            start = base + c * cs
            pltpu.sync_copy(p0_hbm.at[pl.ds(start, cs)], i0)
            pltpu.sync_copy(p1_hbm.at[pl.ds(start, cs)], i1)
            pltpu.async_copy(ys_hbm.at[i0], r0, sem).wait()
            pltpu.async_copy(ys_hbm.at[i1], r1, sem).wait()

            def radd(i, c2):
                for j in range(_D // _L):
                    sl = pl.ds(j * _L, _L)
                    r0[i, sl] = r0[i, sl] + r1[i, sl]
                return c2

            lax.fori_loop(0, cs, radd, 0)
            pltpu.sync_copy(r0, out_hbm.at[pl.ds(start, cs)])
            return carry

        lax.fori_loop(0, tpw // cs, body, 0)

    return k(ys, p0, p1)


def _dispatch_metadata(G, S):
    a_idx = jnp.nonzero(S.reshape(-1) != 0, size=_A, fill_value=0)[0]
    a_idx = a_idx.astype(jnp.int32)
    tok = a_idx // _E
    expert = a_idx % _E
    wgt = G.reshape(-1)[a_idx]
    order = jnp.argsort(expert).astype(jnp.int32)
    tok_s = tok[order]
    wgt_s = wgt[order]
    exp_s = expert[order]
    counts = jnp.bincount(expert, length=_E).astype(jnp.int32)
    nb_e = (counts + _B - 1) // _B
    pad_start = (jnp.concatenate([jnp.zeros(1, jnp.int32),
                                  jnp.cumsum(nb_e)])[:_E] * _B)
    offs = jnp.concatenate([jnp.zeros(1, jnp.int32),
                            jnp.cumsum(counts)])[:_E]
    rank = jnp.arange(_A, dtype=jnp.int32) - offs[exp_s]
    dst = (pad_start[exp_s] + rank).astype(jnp.int32)
    src_tok = jnp.zeros((_NP,), jnp.int32).at[dst].set(tok_s)
    w_row = jnp.zeros((_NP,), jnp.float32).at[dst].set(wgt_s)
    blk_cum = jnp.cumsum(nb_e)
    block_expert = jnp.searchsorted(
        blk_cum, jnp.arange(_NB, dtype=jnp.int32), side="right")
    block_expert = jnp.clip(block_expert, 0, _E - 1).astype(jnp.int32)
    pos = jnp.zeros((_A,), jnp.int32).at[order].set(dst)
    posk = pos.reshape(_T, _K)
    return src_tok, w_row, block_expert, posk[:, 0], posk[:, 1]


def kernel(x, Wg, bg, W1, b1, W2, b2):
    x_flat = x.reshape(_T, _D)
    G, S = _gate(x_flat, Wg, bg.reshape(1, _E))
    src_tok, w_row, block_expert, p0, p1 = _dispatch_metadata(G, S)
    xs = _sc_gather(x_flat, src_tok)
    w_mat = jnp.broadcast_to(w_row[:, None], (_NP, 128))
    ys = _ffn(block_expert, xs, W1, b1.reshape(_E, 1, _F), W2,
              b2.reshape(_E, 1, _D), w_mat)
    out = _sc_combine(ys, p0, p1)
    return out.reshape(x.shape)


# speedup vs baseline: 1.1754x; 1.0984x over previous
import functools

import jax
import jax.numpy as jnp
from jax import lax
from jax.experimental import pallas as pl
from jax.experimental.pallas import tpu as pltpu
from jax.experimental.pallas import tpu_sc as plsc

_D = 1024
_F = 4096
_E = 8
_K = 2
_T = 4096
_A = _T * _K
_B = 128
_NB = _A // _B + _E
_NP = _NB * _B
_TB = 512
_NC = 2
_NS = 16
_NW = _NC * _NS
_L = 16


def _gate_body(x_ref, wg_ref, bg_ref, g_ref, s_ref):
    logits = jnp.dot(x_ref[...], wg_ref[...],
                     preferred_element_type=jnp.float32) + bg_ref[...]
    iota = lax.broadcasted_iota(jnp.int32, logits.shape, 1)
    m1 = jnp.max(logits, axis=1, keepdims=True)
    i1 = jnp.min(jnp.where(logits == m1, iota, _E), axis=1, keepdims=True)
    sel1 = iota == i1
    neg = jnp.float32(float("-inf"))
    l2 = jnp.where(sel1, neg, logits)
    m2 = jnp.max(l2, axis=1, keepdims=True)
    i2 = jnp.min(jnp.where(l2 == m2, iota, _E), axis=1, keepdims=True)
    sel2 = iota == i2
    e21 = jnp.exp(m2 - m1)
    w1 = 1.0 / (1.0 + e21)
    w2 = e21 / (1.0 + e21)
    g_ref[...] = jnp.where(sel1, w1, 0.0) + jnp.where(sel2, w2, 0.0)
    s_ref[...] = (sel1 | sel2).astype(jnp.int32)


def _gate(x_flat, Wg, bg2d):
    return pl.pallas_call(
        _gate_body,
        grid=(_T // _TB,),
        in_specs=[
            pl.BlockSpec((_TB, _D), lambda i: (i, 0)),
            pl.BlockSpec((_D, _E), lambda i: (0, 0)),
            pl.BlockSpec((1, _E), lambda i: (0, 0)),
        ],
        out_specs=[
            pl.BlockSpec((_TB, _E), lambda i: (i, 0)),
            pl.BlockSpec((_TB, _E), lambda i: (i, 0)),
        ],
        out_shape=[
            jax.ShapeDtypeStruct((_T, _E), jnp.float32),
            jax.ShapeDtypeStruct((_T, _E), jnp.int32),
        ],
    )(x_flat, Wg, bg2d)


def _ffn_body(be_ref, bv_ref, xs_ref, w1_ref, b1_ref, w2_ref, b2_ref, wc_ref,
              ys_ref):
    b = pl.program_id(0)

    @pl.when(bv_ref[b] == 1)
    def _compute():
        xb = xs_ref[...].astype(jnp.bfloat16)
        h = jnp.maximum(
            jnp.dot(xb, w1_ref[0],
                    preferred_element_type=jnp.float32) + b1_ref[0], 0.0)
        hb = h.astype(jnp.bfloat16)
        out = jnp.dot(hb, w2_ref[0], preferred_element_type=jnp.float32)
        ys_ref[...] = (out + b2_ref[0]) * wc_ref[...][:, 0:1]


def _ffn(block_expert, block_valid, xs, W1b, b1r, W2b, b2r, w_mat):
    grid_spec = pltpu.PrefetchScalarGridSpec(
        num_scalar_prefetch=2,
        grid=(_NB,),
        in_specs=[
            pl.BlockSpec((_B, _D), lambda b, be, bv: (b, 0)),
            pl.BlockSpec((1, _D, _F), lambda b, be, bv: (be[b], 0, 0)),
            pl.BlockSpec((1, 1, _F), lambda b, be, bv: (be[b], 0, 0)),
            pl.BlockSpec((1, _F, _D), lambda b, be, bv: (be[b], 0, 0)),
            pl.BlockSpec((1, 1, _D), lambda b, be, bv: (be[b], 0, 0)),
            pl.BlockSpec((_B, 128), lambda b, be, bv: (b, 0)),
        ],
        out_specs=pl.BlockSpec((_B, _D), lambda b, be, bv: (b, 0)),
    )
    return pl.pallas_call(
        _ffn_body,
        grid_spec=grid_spec,
        out_shape=jax.ShapeDtypeStruct((_NP, _D), jnp.float32),
    )(block_expert, block_valid, xs, W1b, b1r, W2b, b2r, w_mat)


def _sc_gather(x_flat, src_tok):
    rpw = _NP // _NW
    cs = 48
    nck = rpw // cs
    mesh = plsc.VectorSubcoreMesh(core_axis_name="c", subcore_axis_name="s")

    @functools.partial(
        pl.kernel, mesh=mesh,
        out_type=jax.ShapeDtypeStruct((_NP, _D), jnp.float32),
        scratch_types=[
            pltpu.VMEM((cs,), jnp.int32),
            pltpu.VMEM((cs,), jnp.int32),
            pltpu.VMEM((cs, _D), jnp.float32),
            pltpu.VMEM((cs, _D), jnp.float32),
            pltpu.SemaphoreType.DMA,
            pltpu.SemaphoreType.DMA,
            pltpu.SemaphoreType.DMA,
            pltpu.SemaphoreType.DMA,
        ],
    )
    def k(x_hbm, idx_hbm, out_hbm, idx0, idx1, rows0, rows1,
          gs0, gs1, os0, os1):
        wid = lax.axis_index("s") * _NC + lax.axis_index("c")
        base = wid * rpw
        idxs = (idx0, idx1)
        rows = (rows0, rows1)
        gsem = (gs0, gs1)
        osem = (os0, os1)

        pltpu.sync_copy(idx_hbm.at[pl.ds(base, cs)], idx0)
        gathers = [pltpu.async_copy(x_hbm.at[idx0], rows0, gs0)]
        outs = [None, None]
        for i in range(nck):
            s = i % 2
            if i + 1 < nck:
                ns = (i + 1) % 2
                pltpu.sync_copy(
                    idx_hbm.at[pl.ds(base + (i + 1) * cs, cs)], idxs[ns])
                if outs[ns] is not None:
                    outs[ns].wait()
                gathers.append(
                    pltpu.async_copy(x_hbm.at[idxs[ns]], rows[ns], gsem[ns]))
            gathers[i].wait()
            outs[s] = pltpu.async_copy(
                rows[s], out_hbm.at[pl.ds(base + i * cs, cs)], osem[s])
        outs[0].wait()
        outs[1].wait()

    return k(x_flat, src_tok)


def _sc_combine(ys, p0, p1):
    tpw = _T // _NW
    cs = 32
    mesh = plsc.VectorSubcoreMesh(core_axis_name="c", subcore_axis_name="s")

    @functools.partial(
        pl.kernel, mesh=mesh,
        out_type=jax.ShapeDtypeStruct((_T, _D), jnp.float32),
        scratch_types=[
            pltpu.VMEM((cs,), jnp.int32),
            pltpu.VMEM((cs,), jnp.int32),
            pltpu.VMEM((cs, _D), jnp.float32),
            pltpu.VMEM((cs, _D), jnp.float32),
            pltpu.SemaphoreType.DMA,
        ],
    )
    def k(ys_hbm, p0_hbm, p1_hbm, out_hbm, i0, i1, r0, r1, sem):
        wid = lax.axis_index("s") * _NC + lax.axis_index("c")
        base = wid * tpw

        def body(c, carry):
            start = base + c * cs
            pltpu.sync_copy(p0_hbm.at[pl.ds(start, cs)], i0)
            pltpu.sync_copy(p1_hbm.at[pl.ds(start, cs)], i1)
            pltpu.async_copy(ys_hbm.at[i0], r0, sem).wait()
            pltpu.async_copy(ys_hbm.at[i1], r1, sem).wait()

            def radd(i, c2):
                for j in range(_D // _L):
                    sl = pl.ds(j * _L, _L)
                    r0[i, sl] = r0[i, sl] + r1[i, sl]
                return c2

            lax.fori_loop(0, cs, radd, 0)
            pltpu.sync_copy(r0, out_hbm.at[pl.ds(start, cs)])
            return carry

        lax.fori_loop(0, tpw // cs, body, 0)

    return k(ys, p0, p1)


def _dispatch_metadata(G, S):
    a_idx = jnp.nonzero(S.reshape(-1) != 0, size=_A, fill_value=0)[0]
    a_idx = a_idx.astype(jnp.int32)
    tok = a_idx // _E
    expert = a_idx % _E
    wgt = G.reshape(-1)[a_idx]
    order = jnp.argsort(expert).astype(jnp.int32)
    tok_s = tok[order]
    wgt_s = wgt[order]
    exp_s = expert[order]
    counts = jnp.bincount(expert, length=_E).astype(jnp.int32)
    nb_e = (counts + _B - 1) // _B
    pad_start = (jnp.concatenate([jnp.zeros(1, jnp.int32),
                                  jnp.cumsum(nb_e)])[:_E] * _B)
    offs = jnp.concatenate([jnp.zeros(1, jnp.int32),
                            jnp.cumsum(counts)])[:_E]
    rank = jnp.arange(_A, dtype=jnp.int32) - offs[exp_s]
    dst = (pad_start[exp_s] + rank).astype(jnp.int32)
    src_tok = jnp.zeros((_NP,), jnp.int32).at[dst].set(tok_s)
    w_row = jnp.zeros((_NP,), jnp.float32).at[dst].set(wgt_s)
    blk_cum = jnp.cumsum(nb_e)
    total_blocks = blk_cum[-1]
    block_ids = jnp.arange(_NB, dtype=jnp.int32)
    block_expert = jnp.searchsorted(blk_cum, block_ids, side="right")
    block_expert = jnp.clip(block_expert, 0, _E - 1).astype(jnp.int32)
    block_valid = (block_ids < total_blocks).astype(jnp.int32)
    pos = jnp.zeros((_A,), jnp.int32).at[order].set(dst)
    posk = pos.reshape(_T, _K)
    return src_tok, w_row, block_expert, block_valid, posk[:, 0], posk[:, 1]


def kernel(x, Wg, bg, W1, b1, W2, b2):
    x_flat = x.reshape(_T, _D)
    G, S = _gate(x_flat, Wg, bg.reshape(1, _E))
    src_tok, w_row, block_expert, block_valid, p0, p1 = _dispatch_metadata(G, S)
    xs = _sc_gather(x_flat, src_tok)
    w_mat = jnp.broadcast_to(w_row[:, None], (_NP, 128))
    ys = _ffn(block_expert, block_valid, xs, W1.astype(jnp.bfloat16),
              b1.reshape(_E, 1, _F), W2.astype(jnp.bfloat16),
              b2.reshape(_E, 1, _D), w_mat)
    out = _sc_combine(ys, p0, p1)
    return out.reshape(x.shape)
